# baseline (speedup 1.0000x reference)
import functools

import jax
import jax.numpy as jnp
from jax import lax
from jax.experimental import pallas as pl
from jax.experimental.pallas import tpu as pltpu
from jax.experimental.pallas import tpu_sc as plsc

_CAP = 2
_INT_MIN = -2147483648
_BM = 256


def _monotone_keys(s):
    m = jax.lax.bitcast_convert_type(s, jnp.int32)
    return jnp.where(m >= 0, m, (~m) ^ jnp.int32(_INT_MIN))


def _routing_kernel(scores8_ref, head_ref, code_ref, gate_ref, meta_ref,
                    blk_ref, *, k, nblk):
    s8 = scores8_ref[...]
    E, S = s8.shape
    keys = _monotone_keys(s8)

    cnt_nonneg = jnp.sum((keys >= 0).astype(jnp.int32))
    th0 = jnp.where(cnt_nonneg >= k, jnp.int32(0), jnp.int32(_INT_MIN))

    def body(i, th):
        cand = th | jax.lax.shift_left(jnp.int32(1), jnp.int32(30) - i)
        cnt = jnp.sum((keys >= cand).astype(jnp.int32))
        return jnp.where(cnt >= k, cand, th)

    theta = jax.lax.fori_loop(0, 31, body, th0)
    gt = keys > theta
    eq = keys == theta
    code_ref[...] = 2 * gt.astype(jnp.int32) + eq.astype(jnp.int32)

    head = head_ref[...]
    col = jax.lax.broadcasted_iota(jnp.int32, (E, S), 1)
    gate = jnp.broadcast_to(head[:, E - 1][:, None], (E, S))
    for c in range(E - 1):
        gate = jnp.where(col == c, head[:, c][:, None], gate)
    gate_ref[...] = gate

    gt_cnt = jnp.sum(gt.astype(jnp.float32), axis=1)
    eq_cnt = jnp.sum(eq.astype(jnp.float32), axis=1)
    need = jnp.float32(k) - jnp.sum(gt_cnt)
    row_i = jax.lax.broadcasted_iota(jnp.int32, (E, E), 0)
    col_i = jax.lax.broadcasted_iota(jnp.int32, (E, E), 1)
    excl = (col_i < row_i).astype(jnp.float32)
    eq_excl = jnp.dot(excl, eq_cnt, preferred_element_type=jnp.float32)
    take_eq = jnp.clip(need - eq_excl, 0.0, eq_cnt)
    cnt = (gt_cnt + take_eq).astype(jnp.int32)
    nb = (cnt + (_BM - 1)) // _BM
    off_blk = jnp.dot(excl, nb.astype(jnp.float32),
                      preferred_element_type=jnp.float32).astype(jnp.int32)
    end_blk = off_blk + nb
    total_blk = jnp.sum(nb)

    lane = jax.lax.broadcasted_iota(jnp.int32, (E, 16), 1)
    meta = jnp.where(lane == 0, (off_blk * _BM)[:, None], 0)
    meta = meta + jnp.where(lane == 1, take_eq.astype(jnp.int32)[:, None], 0)
    meta = meta + jnp.where(lane == 2, nb[:, None], 0)
    meta = meta + jnp.where(lane == 3, total_blk, 0)
    meta_ref[...] = meta

    b_i = jax.lax.broadcasted_iota(jnp.int32, (E, 64), 1)
    bex = jnp.sum((b_i >= end_blk[:, None]).astype(jnp.int32), axis=0)
    blk_ref[...] = jnp.broadcast_to(jnp.minimum(bex, E - 1)[None, :], (E, 64))


def _route(scores, k, nblk):
    S, E = scores.shape
    scores8 = scores.reshape(E, S)
    head = scores[:E, :]
    return pl.pallas_call(
        functools.partial(_routing_kernel, k=k, nblk=nblk),
        out_shape=(
            jax.ShapeDtypeStruct((E, S), jnp.int32),
            jax.ShapeDtypeStruct((E, S), jnp.float32),
            jax.ShapeDtypeStruct((E, 16), jnp.int32),
            jax.ShapeDtypeStruct((E, 64), jnp.int32),
        ),
    )(scores8, head)


def _make_compact(E, S, npad, nblk):
    mesh = plsc.VectorSubcoreMesh(core_axis_name="c", subcore_axis_name="s")
    ngrp = S // 16

    @functools.partial(
        pl.kernel, mesh=mesh,
        out_type=(
            jax.ShapeDtypeStruct((npad,), jnp.int32),
            jax.ShapeDtypeStruct((npad,), jnp.float32),
        ),
        scratch_types=[
            pltpu.VMEM((S,), jnp.int32),
            pltpu.VMEM((S,), jnp.float32),
            pltpu.VMEM((S,), jnp.int32),
            pltpu.VMEM((S,), jnp.float32),
            pltpu.VMEM((16,), jnp.int32),
        ],
        compiler_params=pltpu.CompilerParams(needs_layout_passes=False),
    )
    def compact(code_hbm, gate_hbm, meta_hbm, tok_hbm, gout_hbm,
                code_v, gate_v, tok_v, gv_v, meta_v):
        c = lax.axis_index("c")
        s = lax.axis_index("s")
        w = s * 2 + c
        lane = lax.iota(jnp.int32, 16)

        @pl.when(w < E)
        def _():
            pltpu.sync_copy(code_hbm.at[w], code_v)
            pltpu.sync_copy(gate_hbm.at[w], gate_v)
            pltpu.sync_copy(meta_hbm.at[w], meta_v)
            mv = meta_v[...]
            off = jnp.sum(jnp.where(lane == 0, mv, 0))
            take_eq = jnp.sum(jnp.where(lane == 1, mv, 0))
            nb = jnp.sum(jnp.where(lane == 2, mv, 0))

            def zero(i, _):
                tok_v[pl.ds(i * 16, 16)] = jnp.zeros((16,), jnp.int32)
                gv_v[pl.ds(i * 16, 16)] = jnp.zeros((16,), jnp.float32)
                return 0

            lax.fori_loop(0, ngrp, zero, 0)

            def step(g, carry):
                c_sel, c_eq = carry
                cv = code_v[pl.ds(g * 16, 16)]
                gvv = gate_v[pl.ds(g * 16, 16)]
                gtm = cv >= 2
                eqm = cv == 1
                eq_rank = c_eq + plsc.cumsum(eqm.astype(jnp.int32)) - 1
                sel = gtm | (eqm & (eq_rank < take_eq))
                pos = jnp.maximum(
                    c_sel + plsc.cumsum(sel.astype(jnp.int32)) - 1, 0)
                toks = g * 16 + lane
                plsc.store_scatter(tok_v, [pos], toks, mask=sel)
                plsc.store_scatter(gv_v, [pos], gvv, mask=sel)
                return (c_sel + jnp.sum(sel.astype(jnp.int32)),
                        c_eq + jnp.sum(eqm.astype(jnp.int32)))

            lax.fori_loop(0, ngrp, step,
                          (jnp.int32(0), jnp.int32(0)))

            def wb(b, _):
                dst0 = pl.multiple_of(off + b * _BM, _BM)
                src0 = pl.multiple_of(b * _BM, _BM)
                pltpu.sync_copy(tok_v.at[pl.ds(src0, _BM)],
                                tok_hbm.at[pl.ds(dst0, _BM)])
                pltpu.sync_copy(gv_v.at[pl.ds(src0, _BM)],
                                gout_hbm.at[pl.ds(dst0, _BM)])
                return 0

            lax.fori_loop(0, nb, wb, 0)

        @pl.when(w == E)
        def _():
            pltpu.sync_copy(meta_hbm.at[0], meta_v)
            total_blk = jnp.sum(jnp.where(lane == 3, meta_v[...], 0))

            def zero16(i, _):
                tok_v[pl.ds(i * 16, 16)] = jnp.zeros((16,), jnp.int32)
                gv_v[pl.ds(i * 16, 16)] = jnp.zeros((16,), jnp.float32)
                return 0

            lax.fori_loop(0, _BM // 16, zero16, 0)

            def zb(b, _):
                dst0 = pl.multiple_of(b * _BM, _BM)
                pltpu.sync_copy(tok_v.at[pl.ds(0, _BM)],
                                tok_hbm.at[pl.ds(dst0, _BM)])
                pltpu.sync_copy(gv_v.at[pl.ds(0, _BM)],
                                gout_hbm.at[pl.ds(dst0, _BM)])
                return 0

            lax.fori_loop(total_blk, nblk, zb, 0)

    return compact


def _make_gather(S, D, npad):
    mesh = plsc.VectorSubcoreMesh(core_axis_name="c", subcore_axis_name="s")
    rows_per_w = npad // 32
    chunk = 80
    nchunk = rows_per_w // chunk

    @functools.partial(
        pl.kernel, mesh=mesh,
        out_type=jax.ShapeDtypeStruct((npad, D), jnp.float32),
        scratch_types=[
            pltpu.VMEM((chunk,), jnp.int32),
            pltpu.VMEM((chunk,), jnp.int32),
            pltpu.VMEM((chunk, D), jnp.float32),
            pltpu.VMEM((chunk, D), jnp.float32),
            pltpu.SemaphoreType.DMA,
            pltpu.SemaphoreType.DMA,
            pltpu.SemaphoreType.DMA,
            pltpu.SemaphoreType.DMA,
        ],
        compiler_params=pltpu.CompilerParams(needs_layout_passes=False),
    )
    def gather(xf_hbm, tok_hbm, xsel_hbm, idx0, idx1, rows0, rows1,
               g0, g1, w0, w1):
        c = lax.axis_index("c")
        s = lax.axis_index("s")
        w = s * 2 + c
        base = w * rows_per_w
        idx = (idx0, idx1)
        rows = (rows0, rows1)
        gsem = (g0, g1)
        wsem = (w0, w1)
        hg = [None, None]
        hw = [None, None]

        def start(ci):
            b = ci & 1
            if hw[b] is not None:
                hw[b].wait()
                hw[b] = None
            r0 = pl.multiple_of(base + ci * chunk, 8)
            pltpu.sync_copy(tok_hbm.at[pl.ds(r0, chunk)], idx[b])
            hg[b] = pltpu.async_copy(xf_hbm.at[idx[b]], rows[b], gsem[b])

        start(0)
        for ci in range(nchunk):
            b = ci & 1
            if ci + 1 < nchunk:
                start(ci + 1)
            hg[b].wait()
            r0 = pl.multiple_of(base + ci * chunk, 8)
            hw[b] = pltpu.async_copy(rows[b], xsel_hbm.at[pl.ds(r0, chunk)],
                                     wsem[b])
        for b in range(2):
            if hw[b] is not None:
                hw[b].wait()

    return gather


def _ffn_kernel(blk_sref, x_ref, w1_ref, b1_ref, w2_ref, b2_ref, gate_ref,
                y_ref, yacc_ref, *, nf, bm):
    f = pl.program_id(0)
    m = pl.program_id(1)
    row0 = pl.multiple_of(m * bm, bm)

    xb = x_ref[...]
    h = jnp.dot(xb, w1_ref[0], preferred_element_type=jnp.float32)
    h = jax.nn.gelu(h + b1_ref[0], approximate=True)
    part = jnp.dot(h.astype(jnp.bfloat16), w2_ref[0],
                   preferred_element_type=jnp.float32)

    @pl.when(f == 0)
    def _():
        yacc_ref[pl.ds(row0, bm), :] = part.astype(jnp.bfloat16)

    @pl.when(f == nf - 1)
    def _():
        g = gate_ref[0, 0, :]
        prev = yacc_ref[pl.ds(row0, bm), :].astype(jnp.float32)
        y = (prev + part + b2_ref[0]) * g[:, None]
        y_ref[...] = y.astype(jnp.bfloat16)


def _ffn(blk_expert, x_sel, W1, b1, W2, b2, row_gate, npad, nblk, bf):
    E, D, DFF = W1.shape
    nf = DFF // bf
    assert nf == 2
    grid_spec = pltpu.PrefetchScalarGridSpec(
        num_scalar_prefetch=1,
        grid=(nf, nblk),
        in_specs=[
            pl.BlockSpec((_BM, D), lambda f, m, bex: (m, 0)),
            pl.BlockSpec((1, D, bf), lambda f, m, bex: (bex[m], 0, f)),
            pl.BlockSpec((1, 1, bf), lambda f, m, bex: (bex[m], 0, f)),
            pl.BlockSpec((1, bf, D), lambda f, m, bex: (bex[m], f, 0)),
            pl.BlockSpec((1, 1, D), lambda f, m, bex: (bex[m], 0, 0)),
            pl.BlockSpec((1, 1, _BM), lambda f, m, bex: (m, 0, 0)),
        ],
        out_specs=pl.BlockSpec(
            (_BM, D), lambda f, m, bex: (jnp.where(f == nf - 1, m, 0), 0)),
        scratch_shapes=[pltpu.VMEM((npad, D), jnp.bfloat16)],
    )
    return pl.pallas_call(
        functools.partial(_ffn_kernel, nf=nf, bm=_BM),
        grid_spec=grid_spec,
        out_shape=jax.ShapeDtypeStruct((npad, D), jnp.bfloat16),
    )(blk_expert, x_sel, W1, b1.reshape(E, 1, DFF), W2, b2.reshape(E, 1, D),
      row_gate.reshape(nblk, 1, _BM))


def _combine_kernel(y_ref, tok_ref, out_ref, *, nblk, s):
    m = pl.program_id(0)

    @pl.when(m == 0)
    def _():
        out_ref[...] = jnp.zeros_like(out_ref)

    tok = tok_ref[0, 0, :]
    t_i = jax.lax.broadcasted_iota(jnp.int32, (tok.shape[0], s), 1)
    p = (t_i == tok[:, None]).astype(jnp.bfloat16)
    out_ref[...] += jax.lax.dot_general(
        p, y_ref[...], (((0,), (0,)), ((), ())),
        preferred_element_type=jnp.float32)


def _combine(y, row_tok, npad, nblk, S, D):
    return pl.pallas_call(
        functools.partial(_combine_kernel, nblk=nblk, s=S),
        grid=(nblk,),
        in_specs=[
            pl.BlockSpec((_BM, D), lambda m: (m, 0)),
            pl.BlockSpec((1, 1, _BM), lambda m: (m, 0, 0)),
        ],
        out_specs=pl.BlockSpec((S, D), lambda m: (0, 0)),
        out_shape=jax.ShapeDtypeStruct((S, D), jnp.float32),
    )(y, row_tok.reshape(nblk, 1, _BM))


def kernel(x, gate_weight, expert_bias, W1, b1, W2, b2):
    B, S_len, D = x.shape
    E = gate_weight.shape[0]
    DFF = W1.shape[2]
    S = B * S_len
    k = S * _CAP
    npad = k + E * _BM
    nblk = npad // _BM

    xf = x.reshape(S, D)
    logits = (xf @ gate_weight.T).T
    logits = logits + expert_bias
    scores = jax.nn.softmax(logits, axis=0).T

    code8, gate8, meta, blk = _route(scores, k, nblk)
    row_tok, row_gate = _make_compact(E, S, npad, nblk)(code8, gate8, meta)

    xf16 = xf.astype(jnp.bfloat16)
    xview = jax.lax.bitcast_convert_type(
        xf16.reshape(S, D // 2, 2), jnp.float32)
    xsel_view = _make_gather(S, D // 2, npad)(xview, row_tok)
    x_sel = jax.lax.bitcast_convert_type(
        xsel_view, jnp.bfloat16).reshape(npad, D)

    y = _ffn(blk[0, :nblk], x_sel, W1.astype(jnp.bfloat16), b1,
             W2.astype(jnp.bfloat16), b2, row_gate, npad, nblk, DFF // 2)
    out = _combine(y, row_tok, npad, nblk, S, D)
    return (out.reshape(B, S_len, D), None, None)


# speedup vs baseline: 1.5689x; 1.5689x over previous
import functools

import jax
import jax.numpy as jnp
from jax import lax
from jax.experimental import pallas as pl
from jax.experimental.pallas import tpu as pltpu
from jax.experimental.pallas import tpu_sc as plsc

_CAP = 2
_INT_MIN = -2147483648
_BM = 256


def _monotone_keys(s):
    m = jax.lax.bitcast_convert_type(s, jnp.int32)
    return jnp.where(m >= 0, m, (~m) ^ jnp.int32(_INT_MIN))


def _routing_kernel(scores8_ref, head_ref, code_ref, gate_ref, meta_ref,
                    blk_ref, *, k, nblk):
    s8 = scores8_ref[...]
    E, S = s8.shape
    keys = _monotone_keys(s8)

    cnt_nonneg = jnp.sum((keys >= 0).astype(jnp.int32))
    th0 = jnp.where(cnt_nonneg >= k, jnp.int32(0), jnp.int32(_INT_MIN))

    def body(i, th):
        cand = th | jax.lax.shift_left(jnp.int32(1), jnp.int32(30) - i)
        cnt = jnp.sum((keys >= cand).astype(jnp.int32))
        return jnp.where(cnt >= k, cand, th)

    theta = jax.lax.fori_loop(0, 31, body, th0)
    gt = keys > theta
    eq = keys == theta
    code_ref[...] = 2 * gt.astype(jnp.int32) + eq.astype(jnp.int32)

    head = head_ref[...]
    col = jax.lax.broadcasted_iota(jnp.int32, (E, S), 1)
    gate = jnp.broadcast_to(head[:, E - 1][:, None], (E, S))
    for c in range(E - 1):
        gate = jnp.where(col == c, head[:, c][:, None], gate)
    gate_ref[...] = gate

    gt_cnt = jnp.sum(gt.astype(jnp.float32), axis=1)
    eq_cnt = jnp.sum(eq.astype(jnp.float32), axis=1)
    need = jnp.float32(k) - jnp.sum(gt_cnt)
    row_i = jax.lax.broadcasted_iota(jnp.int32, (E, E), 0)
    col_i = jax.lax.broadcasted_iota(jnp.int32, (E, E), 1)
    excl = (col_i < row_i).astype(jnp.float32)
    eq_excl = jnp.dot(excl, eq_cnt, preferred_element_type=jnp.float32)
    take_eq = jnp.clip(need - eq_excl, 0.0, eq_cnt)
    cnt = (gt_cnt + take_eq).astype(jnp.int32)
    nb = (cnt + (_BM - 1)) // _BM
    off_blk = jnp.dot(excl, nb.astype(jnp.float32),
                      preferred_element_type=jnp.float32).astype(jnp.int32)
    end_blk = off_blk + nb
    total_blk = jnp.sum(nb)

    lane = jax.lax.broadcasted_iota(jnp.int32, (E, 16), 1)
    meta = jnp.where(lane == 0, (off_blk * _BM)[:, None], 0)
    meta = meta + jnp.where(lane == 1, take_eq.astype(jnp.int32)[:, None], 0)
    meta = meta + jnp.where(lane == 2, nb[:, None], 0)
    meta = meta + jnp.where(lane == 3, total_blk, 0)
    meta_ref[...] = meta

    b_i = jax.lax.broadcasted_iota(jnp.int32, (E, 64), 1)
    bex = jnp.sum((b_i >= end_blk[:, None]).astype(jnp.int32), axis=0)
    blk_ref[...] = jnp.broadcast_to(jnp.minimum(bex, E - 1)[None, :], (E, 64))


def _route(scores, k, nblk):
    S, E = scores.shape
    scores8 = scores.reshape(E, S)
    head = scores[:E, :]
    return pl.pallas_call(
        functools.partial(_routing_kernel, k=k, nblk=nblk),
        out_shape=(
            jax.ShapeDtypeStruct((E, S), jnp.int32),
            jax.ShapeDtypeStruct((E, S), jnp.float32),
            jax.ShapeDtypeStruct((E, 16), jnp.int32),
            jax.ShapeDtypeStruct((E, 64), jnp.int32),
        ),
    )(scores8, head)


def _make_compact(E, S, npad, nblk):
    mesh = plsc.VectorSubcoreMesh(core_axis_name="c", subcore_axis_name="s")
    ngrp = S // 16

    @functools.partial(
        pl.kernel, mesh=mesh,
        out_type=(
            jax.ShapeDtypeStruct((npad,), jnp.int32),
            jax.ShapeDtypeStruct((npad,), jnp.float32),
        ),
        scratch_types=[
            pltpu.VMEM((S,), jnp.int32),
            pltpu.VMEM((S,), jnp.float32),
            pltpu.VMEM((S,), jnp.int32),
            pltpu.VMEM((S,), jnp.float32),
            pltpu.VMEM((16,), jnp.int32),
        ],
        compiler_params=pltpu.CompilerParams(needs_layout_passes=False),
    )
    def compact(code_hbm, gate_hbm, meta_hbm, tok_hbm, gout_hbm,
                code_v, gate_v, tok_v, gv_v, meta_v):
        c = lax.axis_index("c")
        s = lax.axis_index("s")
        w = s * 2 + c
        lane = lax.iota(jnp.int32, 16)

        @pl.when(w < E)
        def _():
            pltpu.sync_copy(code_hbm.at[w], code_v)
            pltpu.sync_copy(gate_hbm.at[w], gate_v)
            pltpu.sync_copy(meta_hbm.at[w], meta_v)
            mv = meta_v[...]
            off = jnp.sum(jnp.where(lane == 0, mv, 0))
            take_eq = jnp.sum(jnp.where(lane == 1, mv, 0))
            nb = jnp.sum(jnp.where(lane == 2, mv, 0))

            def zero(i, _):
                tok_v[pl.ds(i * 16, 16)] = jnp.zeros((16,), jnp.int32)
                gv_v[pl.ds(i * 16, 16)] = jnp.zeros((16,), jnp.float32)
                return 0

            lax.fori_loop(0, ngrp, zero, 0)

            def step(g, carry):
                c_sel, c_eq = carry
                cv = code_v[pl.ds(g * 16, 16)]
                gvv = gate_v[pl.ds(g * 16, 16)]
                gtm = cv >= 2
                eqm = cv == 1
                eq_rank = c_eq + plsc.cumsum(eqm.astype(jnp.int32)) - 1
                sel = gtm | (eqm & (eq_rank < take_eq))
                pos = jnp.maximum(
                    c_sel + plsc.cumsum(sel.astype(jnp.int32)) - 1, 0)
                toks = g * 16 + lane
                plsc.store_scatter(tok_v, [pos], toks, mask=sel)
                plsc.store_scatter(gv_v, [pos], gvv, mask=sel)
                return (c_sel + jnp.sum(sel.astype(jnp.int32)),
                        c_eq + jnp.sum(eqm.astype(jnp.int32)))

            lax.fori_loop(0, ngrp, step,
                          (jnp.int32(0), jnp.int32(0)))

            def wb(b, _):
                dst0 = pl.multiple_of(off + b * _BM, _BM)
                src0 = pl.multiple_of(b * _BM, _BM)
                pltpu.sync_copy(tok_v.at[pl.ds(src0, _BM)],
                                tok_hbm.at[pl.ds(dst0, _BM)])
                pltpu.sync_copy(gv_v.at[pl.ds(src0, _BM)],
                                gout_hbm.at[pl.ds(dst0, _BM)])
                return 0

            lax.fori_loop(0, nb, wb, 0)

        @pl.when(w == E)
        def _():
            pltpu.sync_copy(meta_hbm.at[0], meta_v)
            total_blk = jnp.sum(jnp.where(lane == 3, meta_v[...], 0))

            def zero16(i, _):
                tok_v[pl.ds(i * 16, 16)] = jnp.zeros((16,), jnp.int32)
                gv_v[pl.ds(i * 16, 16)] = jnp.zeros((16,), jnp.float32)
                return 0

            lax.fori_loop(0, _BM // 16, zero16, 0)

            def zb(b, _):
                dst0 = pl.multiple_of(b * _BM, _BM)
                pltpu.sync_copy(tok_v.at[pl.ds(0, _BM)],
                                tok_hbm.at[pl.ds(dst0, _BM)])
                pltpu.sync_copy(gv_v.at[pl.ds(0, _BM)],
                                gout_hbm.at[pl.ds(dst0, _BM)])
                return 0

            lax.fori_loop(total_blk, nblk, zb, 0)

    return compact


def _make_gather(S, D, npad):
    mesh = plsc.VectorSubcoreMesh(core_axis_name="c", subcore_axis_name="s")
    rows_per_w = npad // 32
    chunk = 40
    nchunk = rows_per_w // chunk

    @functools.partial(
        pl.kernel, mesh=mesh,
        out_type=jax.ShapeDtypeStruct((npad, D), jnp.float32),
        scratch_types=[
            pltpu.VMEM((chunk,), jnp.int32),
            pltpu.VMEM((chunk,), jnp.int32),
            pltpu.VMEM((chunk, D), jnp.float32),
            pltpu.VMEM((chunk, D), jnp.float32),
            pltpu.SemaphoreType.DMA,
            pltpu.SemaphoreType.DMA,
            pltpu.SemaphoreType.DMA,
            pltpu.SemaphoreType.DMA,
        ],
        compiler_params=pltpu.CompilerParams(needs_layout_passes=False),
    )
    def gather(xf_hbm, tok_hbm, xsel_hbm, idx0, idx1, rows0, rows1,
               g0, g1, w0, w1):
        c = lax.axis_index("c")
        s = lax.axis_index("s")
        w = s * 2 + c
        base = w * rows_per_w
        idx = (idx0, idx1)
        rows = (rows0, rows1)
        gsem = (g0, g1)
        wsem = (w0, w1)
        hg = [None, None]
        hw = [None, None]

        def start(ci):
            b = ci & 1
            if hw[b] is not None:
                hw[b].wait()
                hw[b] = None
            r0 = pl.multiple_of(base + ci * chunk, 8)
            pltpu.sync_copy(tok_hbm.at[pl.ds(r0, chunk)], idx[b])
            hg[b] = pltpu.async_copy(xf_hbm.at[idx[b]], rows[b], gsem[b])

        start(0)
        for ci in range(nchunk):
            b = ci & 1
            if ci + 1 < nchunk:
                start(ci + 1)
            hg[b].wait()
            r0 = pl.multiple_of(base + ci * chunk, 8)
            hw[b] = pltpu.async_copy(rows[b], xsel_hbm.at[pl.ds(r0, chunk)],
                                     wsem[b])
        for b in range(2):
            if hw[b] is not None:
                hw[b].wait()

    return gather


def _ffn_kernel(blk_sref, x_ref, w1_ref, b1_ref, w2_ref, b2_ref, gate_ref,
                y_ref, yacc_ref, *, nf, bm):
    f = pl.program_id(0)
    m = pl.program_id(1)
    row0 = pl.multiple_of(m * bm, bm)

    xb = x_ref[...].astype(jnp.bfloat16)
    w1 = w1_ref[0].astype(jnp.bfloat16)
    h = jnp.dot(xb, w1, preferred_element_type=jnp.float32)
    h = jax.nn.gelu(h + b1_ref[0], approximate=True)
    w2 = w2_ref[0].astype(jnp.bfloat16)
    part = jnp.dot(h.astype(jnp.bfloat16), w2,
                   preferred_element_type=jnp.float32)

    @pl.when(f == 0)
    def _():
        yacc_ref[pl.ds(row0, bm), :] = part.astype(jnp.bfloat16)

    @pl.when(f == nf - 1)
    def _():
        g = gate_ref[0, 0, :]
        prev = yacc_ref[pl.ds(row0, bm), :].astype(jnp.float32)
        y = (prev + part + b2_ref[0]) * g[:, None]
        y_ref[...] = y.astype(jnp.bfloat16)


def _ffn(blk_expert, x_sel, W1, b1, W2, b2, row_gate, npad, nblk, bf):
    E, D, DFF = W1.shape
    nf = DFF // bf
    assert nf == 2
    grid_spec = pltpu.PrefetchScalarGridSpec(
        num_scalar_prefetch=1,
        grid=(nf, nblk),
        in_specs=[
            pl.BlockSpec((_BM, D), lambda f, m, bex: (m, 0)),
            pl.BlockSpec((1, D, bf), lambda f, m, bex: (bex[m], 0, f)),
            pl.BlockSpec((1, 1, bf), lambda f, m, bex: (bex[m], 0, f)),
            pl.BlockSpec((1, bf, D), lambda f, m, bex: (bex[m], f, 0)),
            pl.BlockSpec((1, 1, D), lambda f, m, bex: (bex[m], 0, 0)),
            pl.BlockSpec((1, 1, _BM), lambda f, m, bex: (m, 0, 0)),
        ],
        out_specs=pl.BlockSpec(
            (_BM, D), lambda f, m, bex: (jnp.where(f == nf - 1, m, 0), 0)),
        scratch_shapes=[pltpu.VMEM((npad, D), jnp.bfloat16)],
    )
    return pl.pallas_call(
        functools.partial(_ffn_kernel, nf=nf, bm=_BM),
        grid_spec=grid_spec,
        out_shape=jax.ShapeDtypeStruct((npad, D), jnp.bfloat16),
    )(blk_expert, x_sel, W1, b1.reshape(E, 1, DFF), W2, b2.reshape(E, 1, D),
      row_gate.reshape(nblk, 1, _BM))


def _combine_kernel(y_ref, tok_ref, out_ref, *, nblk, s):
    m = pl.program_id(0)

    @pl.when(m == 0)
    def _():
        out_ref[...] = jnp.zeros_like(out_ref)

    tok = tok_ref[0, 0, :]
    t_i = jax.lax.broadcasted_iota(jnp.int32, (tok.shape[0], s), 1)
    p = (t_i == tok[:, None]).astype(jnp.bfloat16)
    out_ref[...] += jax.lax.dot_general(
        p, y_ref[...], (((0,), (0,)), ((), ())),
        preferred_element_type=jnp.float32)


def _combine(y, row_tok, npad, nblk, S, D):
    return pl.pallas_call(
        functools.partial(_combine_kernel, nblk=nblk, s=S),
        grid=(nblk,),
        in_specs=[
            pl.BlockSpec((_BM, D), lambda m: (m, 0)),
            pl.BlockSpec((1, 1, _BM), lambda m: (m, 0, 0)),
        ],
        out_specs=pl.BlockSpec((S, D), lambda m: (0, 0)),
        out_shape=jax.ShapeDtypeStruct((S, D), jnp.float32),
    )(y, row_tok.reshape(nblk, 1, _BM))


def kernel(x, gate_weight, expert_bias, W1, b1, W2, b2):
    B, S_len, D = x.shape
    E = gate_weight.shape[0]
    DFF = W1.shape[2]
    S = B * S_len
    k = S * _CAP
    npad = k + E * _BM
    nblk = npad // _BM

    xf = x.reshape(S, D)
    logits = (xf @ gate_weight.T).T
    logits = logits + expert_bias
    scores = jax.nn.softmax(logits, axis=0).T

    code8, gate8, meta, blk = _route(scores, k, nblk)
    row_tok, row_gate = _make_compact(E, S, npad, nblk)(code8, gate8, meta)
    x_sel = _make_gather(S, D, npad)(xf, row_tok)
    y = _ffn(blk[0, :nblk], x_sel, W1, b1, W2, b2, row_gate, npad, nblk,
             DFF // 2)
    out = _combine(y, row_tok, npad, nblk, S, D)
    return (out.reshape(B, S_len, D), None, None)


# speedup vs baseline: 1.6103x; 1.0264x over previous
import functools

import jax
import jax.numpy as jnp
from jax import lax
from jax.experimental import pallas as pl
from jax.experimental.pallas import tpu as pltpu
from jax.experimental.pallas import tpu_sc as plsc

_CAP = 2
_INT_MIN = -2147483648
_BM = 256


def _monotone_keys(s):
    m = jax.lax.bitcast_convert_type(s, jnp.int32)
    return jnp.where(m >= 0, m, (~m) ^ jnp.int32(_INT_MIN))


def _routing_kernel(scores8_ref, head_ref, code_ref, gate_ref, meta_ref,
                    blk_ref, *, k, nblk):
    s8 = scores8_ref[...]
    E, S = s8.shape
    keys = _monotone_keys(s8)

    cnt_nonneg = jnp.sum((keys >= 0).astype(jnp.int32))
    th0 = jnp.where(cnt_nonneg >= k, jnp.int32(0), jnp.int32(_INT_MIN))

    def body(i, th):
        cand = th | jax.lax.shift_left(jnp.int32(1), jnp.int32(30) - i)
        cnt = jnp.sum((keys >= cand).astype(jnp.int32))
        return jnp.where(cnt >= k, cand, th)

    theta = jax.lax.fori_loop(0, 31, body, th0)
    gt = keys > theta
    eq = keys == theta
    code_ref[...] = 2 * gt.astype(jnp.int32) + eq.astype(jnp.int32)

    head = head_ref[...]
    col = jax.lax.broadcasted_iota(jnp.int32, (E, S), 1)
    gate = jnp.broadcast_to(head[:, E - 1][:, None], (E, S))
    for c in range(E - 1):
        gate = jnp.where(col == c, head[:, c][:, None], gate)
    gate_ref[...] = gate

    gt_cnt = jnp.sum(gt.astype(jnp.float32), axis=1)
    eq_cnt = jnp.sum(eq.astype(jnp.float32), axis=1)
    need = jnp.float32(k) - jnp.sum(gt_cnt)
    row_i = jax.lax.broadcasted_iota(jnp.int32, (E, E), 0)
    col_i = jax.lax.broadcasted_iota(jnp.int32, (E, E), 1)
    excl = (col_i < row_i).astype(jnp.float32)
    eq_excl = jnp.dot(excl, eq_cnt, preferred_element_type=jnp.float32)
    take_eq = jnp.clip(need - eq_excl, 0.0, eq_cnt)
    cnt = (gt_cnt + take_eq).astype(jnp.int32)
    nb = (cnt + (_BM - 1)) // _BM
    off_blk = jnp.dot(excl, nb.astype(jnp.float32),
                      preferred_element_type=jnp.float32).astype(jnp.int32)
    end_blk = off_blk + nb
    total_blk = jnp.sum(nb)

    lane = jax.lax.broadcasted_iota(jnp.int32, (E, 16), 1)
    meta = jnp.where(lane == 0, (off_blk * _BM)[:, None], 0)
    meta = meta + jnp.where(lane == 1, take_eq.astype(jnp.int32)[:, None], 0)
    meta = meta + jnp.where(lane == 2, nb[:, None], 0)
    meta = meta + jnp.where(lane == 3, total_blk, 0)
    meta_ref[...] = meta

    b_i = jax.lax.broadcasted_iota(jnp.int32, (E, 64), 1)
    bex = jnp.sum((b_i >= end_blk[:, None]).astype(jnp.int32), axis=0)
    blk_ref[...] = jnp.broadcast_to(jnp.minimum(bex, E - 1)[None, :], (E, 64))


def _route(scores, k, nblk):
    S, E = scores.shape
    scores8 = scores.reshape(E, S)
    head = scores[:E, :]
    return pl.pallas_call(
        functools.partial(_routing_kernel, k=k, nblk=nblk),
        out_shape=(
            jax.ShapeDtypeStruct((E, S), jnp.int32),
            jax.ShapeDtypeStruct((E, S), jnp.float32),
            jax.ShapeDtypeStruct((E, 16), jnp.int32),
            jax.ShapeDtypeStruct((E, 64), jnp.int32),
        ),
    )(scores8, head)


def _make_compact(E, S, npad, nblk):
    mesh = plsc.VectorSubcoreMesh(core_axis_name="c", subcore_axis_name="s")
    ngrp = S // 16

    @functools.partial(
        pl.kernel, mesh=mesh,
        out_type=(
            jax.ShapeDtypeStruct((npad,), jnp.int32),
            jax.ShapeDtypeStruct((npad,), jnp.float32),
        ),
        scratch_types=[
            pltpu.VMEM((S,), jnp.int32),
            pltpu.VMEM((S,), jnp.float32),
            pltpu.VMEM((S,), jnp.int32),
            pltpu.VMEM((S,), jnp.float32),
            pltpu.VMEM((16,), jnp.int32),
        ],
        compiler_params=pltpu.CompilerParams(needs_layout_passes=False),
    )
    def compact(code_hbm, gate_hbm, meta_hbm, tok_hbm, gout_hbm,
                code_v, gate_v, tok_v, gv_v, meta_v):
        c = lax.axis_index("c")
        s = lax.axis_index("s")
        w = s * 2 + c
        lane = lax.iota(jnp.int32, 16)

        @pl.when(w < E)
        def _():
            pltpu.sync_copy(code_hbm.at[w], code_v)
            pltpu.sync_copy(gate_hbm.at[w], gate_v)
            pltpu.sync_copy(meta_hbm.at[w], meta_v)
            mv = meta_v[...]
            off = jnp.sum(jnp.where(lane == 0, mv, 0))
            take_eq = jnp.sum(jnp.where(lane == 1, mv, 0))
            nb = jnp.sum(jnp.where(lane == 2, mv, 0))

            def zero(i, _):
                tok_v[pl.ds(i * 16, 16)] = jnp.zeros((16,), jnp.int32)
                gv_v[pl.ds(i * 16, 16)] = jnp.zeros((16,), jnp.float32)
                return 0

            lax.fori_loop(0, ngrp, zero, 0)

            def step(g, carry):
                c_sel, c_eq = carry
                cv = code_v[pl.ds(g * 16, 16)]
                gvv = gate_v[pl.ds(g * 16, 16)]
                gtm = cv >= 2
                eqm = cv == 1
                eq_rank = c_eq + plsc.cumsum(eqm.astype(jnp.int32)) - 1
                sel = gtm | (eqm & (eq_rank < take_eq))
                pos = jnp.maximum(
                    c_sel + plsc.cumsum(sel.astype(jnp.int32)) - 1, 0)
                toks = g * 16 + lane
                plsc.store_scatter(tok_v, [pos], toks, mask=sel)
                plsc.store_scatter(gv_v, [pos], gvv, mask=sel)
                return (c_sel + jnp.sum(sel.astype(jnp.int32)),
                        c_eq + jnp.sum(eqm.astype(jnp.int32)))

            lax.fori_loop(0, ngrp, step,
                          (jnp.int32(0), jnp.int32(0)))

            def wb(b, _):
                dst0 = pl.multiple_of(off + b * _BM, _BM)
                src0 = pl.multiple_of(b * _BM, _BM)
                pltpu.sync_copy(tok_v.at[pl.ds(src0, _BM)],
                                tok_hbm.at[pl.ds(dst0, _BM)])
                pltpu.sync_copy(gv_v.at[pl.ds(src0, _BM)],
                                gout_hbm.at[pl.ds(dst0, _BM)])
                return 0

            lax.fori_loop(0, nb, wb, 0)

        @pl.when(w == E)
        def _():
            pltpu.sync_copy(meta_hbm.at[0], meta_v)
            total_blk = jnp.sum(jnp.where(lane == 3, meta_v[...], 0))

            def zero16(i, _):
                tok_v[pl.ds(i * 16, 16)] = jnp.zeros((16,), jnp.int32)
                gv_v[pl.ds(i * 16, 16)] = jnp.zeros((16,), jnp.float32)
                return 0

            lax.fori_loop(0, _BM // 16, zero16, 0)

            def zb(b, _):
                dst0 = pl.multiple_of(b * _BM, _BM)
                pltpu.sync_copy(tok_v.at[pl.ds(0, _BM)],
                                tok_hbm.at[pl.ds(dst0, _BM)])
                pltpu.sync_copy(gv_v.at[pl.ds(0, _BM)],
                                gout_hbm.at[pl.ds(dst0, _BM)])
                return 0

            lax.fori_loop(total_blk, nblk, zb, 0)

    return compact


def _make_gather(S, D, nrows):
    mesh = plsc.VectorSubcoreMesh(core_axis_name="c", subcore_axis_name="s")
    rows_per_w = nrows // 32
    chunk = 40
    nchunk = rows_per_w // chunk

    @functools.partial(
        pl.kernel, mesh=mesh,
        out_type=jax.ShapeDtypeStruct((nrows, D), jnp.float32),
        scratch_types=[
            pltpu.VMEM((chunk,), jnp.int32),
            pltpu.VMEM((chunk,), jnp.int32),
            pltpu.VMEM((chunk, D), jnp.float32),
            pltpu.VMEM((chunk, D), jnp.float32),
            pltpu.SemaphoreType.DMA,
            pltpu.SemaphoreType.DMA,
            pltpu.SemaphoreType.DMA,
            pltpu.SemaphoreType.DMA,
        ],
        compiler_params=pltpu.CompilerParams(needs_layout_passes=False),
    )
    def gather(xf_hbm, tok_hbm, xsel_hbm, idx0, idx1, rows0, rows1,
               g0, g1, w0, w1):
        c = lax.axis_index("c")
        s = lax.axis_index("s")
        w = s * 2 + c
        base = w * rows_per_w
        idx = (idx0, idx1)
        rows = (rows0, rows1)
        gsem = (g0, g1)
        wsem = (w0, w1)
        hg = [None, None]
        hw = [None, None]

        def start(ci):
            b = ci & 1
            if hw[b] is not None:
                hw[b].wait()
                hw[b] = None
            r0 = pl.multiple_of(base + ci * chunk, 8)
            pltpu.sync_copy(tok_hbm.at[pl.ds(r0, chunk)], idx[b])
            hg[b] = pltpu.async_copy(xf_hbm.at[idx[b]], rows[b], gsem[b])

        start(0)
        for ci in range(nchunk):
            b = ci & 1
            if ci + 1 < nchunk:
                start(ci + 1)
            hg[b].wait()
            r0 = pl.multiple_of(base + ci * chunk, 8)
            hw[b] = pltpu.async_copy(rows[b], xsel_hbm.at[pl.ds(r0, chunk)],
                                     wsem[b])
        for b in range(2):
            if hw[b] is not None:
                hw[b].wait()

    return gather


def _ffn_kernel(blk_sref, x_ref, w1_ref, b1_ref, w2_ref, b2_ref, gate_ref,
                y_ref, yacc_ref, *, nf, bm):
    f = pl.program_id(0)
    m = pl.program_id(1)
    row0 = pl.multiple_of(m * bm, bm)

    xb = x_ref[...].astype(jnp.bfloat16)
    w1 = w1_ref[0].astype(jnp.bfloat16)
    h = jnp.dot(xb, w1, preferred_element_type=jnp.float32)
    h = jax.nn.gelu(h + b1_ref[0], approximate=True)
    w2 = w2_ref[0].astype(jnp.bfloat16)
    part = jnp.dot(h.astype(jnp.bfloat16), w2,
                   preferred_element_type=jnp.float32)

    @pl.when(f == 0)
    def _():
        yacc_ref[pl.ds(row0, bm), :] = part.astype(jnp.bfloat16)

    @pl.when(f == nf - 1)
    def _():
        g = gate_ref[0, 0, :]
        prev = yacc_ref[pl.ds(row0, bm), :].astype(jnp.float32)
        y = (prev + part + b2_ref[0]) * g[:, None]
        y_ref[...] = y.astype(jnp.bfloat16)


def _ffn(blk_expert, x_sel, W1, b1, W2, b2, row_gate, npad, nblk, bf):
    E, D, DFF = W1.shape
    nf = DFF // bf
    assert nf == 2
    grid_spec = pltpu.PrefetchScalarGridSpec(
        num_scalar_prefetch=1,
        grid=(nf, nblk),
        in_specs=[
            pl.BlockSpec((_BM, D), lambda f, m, bex: (m, 0)),
            pl.BlockSpec((1, D, bf), lambda f, m, bex: (bex[m], 0, f)),
            pl.BlockSpec((1, 1, bf), lambda f, m, bex: (bex[m], 0, f)),
            pl.BlockSpec((1, bf, D), lambda f, m, bex: (bex[m], f, 0)),
            pl.BlockSpec((1, 1, D), lambda f, m, bex: (bex[m], 0, 0)),
            pl.BlockSpec((1, 1, _BM), lambda f, m, bex: (m, 0, 0)),
        ],
        out_specs=pl.BlockSpec(
            (_BM, D), lambda f, m, bex: (jnp.where(f == nf - 1, m, 0), 0)),
        scratch_shapes=[pltpu.VMEM((npad, D), jnp.bfloat16)],
    )
    return pl.pallas_call(
        functools.partial(_ffn_kernel, nf=nf, bm=_BM),
        grid_spec=grid_spec,
        out_shape=jax.ShapeDtypeStruct((npad, D), jnp.bfloat16),
    )(blk_expert, x_sel, W1, b1.reshape(E, 1, DFF), W2, b2.reshape(E, 1, D),
      row_gate)


def _combine_kernel(ya_ref, yb_ref, tok_ref, out_ref, *, nhalf, s):
    m = pl.program_id(0)

    @pl.when(m == 0)
    def _():
        out_ref[...] = jnp.zeros_like(out_ref)

    tok = tok_ref[0, 0, :]
    y = jnp.where(m < nhalf, ya_ref[...].astype(jnp.float32),
                  yb_ref[...].astype(jnp.float32)).astype(jnp.bfloat16)
    t_i = jax.lax.broadcasted_iota(jnp.int32, (tok.shape[0], s), 1)
    p = (t_i == tok[:, None]).astype(jnp.bfloat16)
    out_ref[...] += jax.lax.dot_general(
        p, y, (((0,), (0,)), ((), ())),
        preferred_element_type=jnp.float32)


def _combine(y_a, y_b, row_tok, nblk, S, D):
    nhalf = nblk // 2
    return pl.pallas_call(
        functools.partial(_combine_kernel, nhalf=nhalf, s=S),
        grid=(nblk,),
        in_specs=[
            pl.BlockSpec((_BM, D), lambda m: (jnp.minimum(m, nblk // 2 - 1), 0)),
            pl.BlockSpec((_BM, D),
                         lambda m: (jnp.maximum(m - nblk // 2, 0), 0)),
            pl.BlockSpec((1, 1, _BM), lambda m: (m, 0, 0)),
        ],
        out_specs=pl.BlockSpec((S, D), lambda m: (0, 0)),
        out_shape=jax.ShapeDtypeStruct((S, D), jnp.float32),
    )(y_a, y_b, row_tok.reshape(nblk, 1, _BM))


def kernel(x, gate_weight, expert_bias, W1, b1, W2, b2):
    B, S_len, D = x.shape
    E = gate_weight.shape[0]
    DFF = W1.shape[2]
    S = B * S_len
    k = S * _CAP
    npad = k + E * _BM
    nblk = npad // _BM

    xf = x.reshape(S, D)
    logits = (xf @ gate_weight.T).T
    logits = logits + expert_bias
    scores = jax.nn.softmax(logits, axis=0).T

    code8, gate8, meta, blk = _route(scores, k, nblk)
    row_tok, row_gate = _make_compact(E, S, npad, nblk)(code8, gate8, meta)

    nh = npad // 2
    nbh = nblk // 2
    gather = _make_gather(S, D, nh)
    gate3 = row_gate.reshape(nblk, 1, _BM)
    halves = []
    for hidx in (0, 1):
        r0, b0 = hidx * nh, hidx * nbh
        x_sel = gather(xf, jax.lax.dynamic_slice_in_dim(row_tok, r0, nh))
        y = _ffn(blk[0, b0:b0 + nbh], x_sel, W1, b1, W2, b2,
                 gate3[b0:b0 + nbh], nh, nbh, DFF // 2)
        halves.append(y)
    out = _combine(halves[0], halves[1], row_tok, nblk, S, D)
    return (out.reshape(B, S_len, D), None, None)
